# Initial kernel scaffold; baseline (speedup 1.0000x reference)
import jax
import jax.numpy as jnp
from jax.experimental import pallas as pl


def kernel(x, edge_index, W1, b1, W2, b2):
    raise NotImplementedError("write your pallas kernel here")


# speedup vs baseline: 17.3305x; 17.3305x over previous
import functools

import jax
import jax.numpy as jnp
from jax import lax
from jax.experimental import pallas as pl
from jax.experimental.pallas import tpu as pltpu
from jax.experimental.pallas import tpu_sc as plsc

N = 10000
NPAD = 10240
E = 320000
D = 128
CH = 125
G = 8
NC = 2
NS = 16
RPT = NPAD // NS

_mesh = plsc.VectorSubcoreMesh(core_axis_name="c", subcore_axis_name="s")
_f32 = jnp.float32


@functools.partial(
    pl.kernel,
    mesh=_mesh,
    out_type=[jax.ShapeDtypeStruct((NPAD,), _f32),
              jax.ShapeDtypeStruct((NPAD,), _f32)],
    scratch_types=[
        pltpu.VMEM((E // (NC * NS * CH), CH), jnp.int32),
        pltpu.VMEM((128,), _f32),
        pltpu.VMEM((RPT,), _f32),
        pltpu.VMEM_SHARED((NPAD,), _f32),
    ],
)
def _deg_kernel(dst_hbm, deg_a, deg_b, dst_v, ones_v, zero_v, acc):
    c = lax.axis_index("c")
    s = lax.axis_index("s")
    rows = E // (NC * NS * CH)
    wid = s * NC + c

    def fill_ones(i, _):
        ones_v[pl.ds(i * 16, 16)] = jnp.ones((16,), _f32)
        return 0

    lax.fori_loop(0, 8, fill_ones, 0)

    def fill_zero(i, _):
        zero_v[pl.ds(i * 16, 16)] = jnp.zeros((16,), _f32)
        return 0

    lax.fori_loop(0, RPT // 16, fill_zero, 0)

    nsl = pl.ds(s * RPT, RPT)
    pltpu.sync_copy(zero_v, acc.at[nsl])
    pltpu.sync_copy(dst_hbm.at[wid], dst_v)
    plsc.subcore_barrier()

    def body(j, _):
        pltpu.sync_copy(ones_v.at[pl.ds(0, CH)], acc.at[dst_v.at[j]], add=True)
        return 0

    lax.fori_loop(0, rows, body, 0)
    plsc.subcore_barrier()

    @pl.when(c == 0)
    def _():
        pltpu.sync_copy(acc.at[nsl], deg_a.at[nsl])

    @pl.when(c == 1)
    def _():
        pltpu.sync_copy(acc.at[nsl], deg_b.at[nsl])


def _make_agg(split_edges_by_core: bool):
    rows = E // (NC * NS * CH) if split_edges_by_core else E // (NS * CH)

    @functools.partial(
        pl.kernel,
        mesh=_mesh,
        out_type=[jax.ShapeDtypeStruct((NPAD, D), _f32),
                  jax.ShapeDtypeStruct((NPAD, D), _f32)],
        scratch_types=[
            pltpu.VMEM((G, CH), jnp.int32),
            pltpu.VMEM((G, CH), jnp.int32),
            pltpu.VMEM((CH, D), _f32),
            pltpu.VMEM_SHARED((NPAD, D), _f32),
            pltpu.SemaphoreType.DMA,
        ],
    )
    def agg(src_hbm, dst_hbm, table_a, table_b, out_a, out_b,
            src_v, dst_v, rows_v, acc, sem):
        c = lax.axis_index("c")
        s = lax.axis_index("s")
        if split_edges_by_core:
            tid = s * NC + c
        else:
            tid = s
        nsl = pl.ds(s * RPT, RPT)

        def run(table, out):
            pltpu.sync_copy(table.at[nsl], acc.at[nsl])
            plsc.subcore_barrier()

            def group(g, _):
                gsl = pl.ds(g * G, G)
                pltpu.sync_copy(src_hbm.at[tid].at[gsl], src_v)
                pltpu.sync_copy(dst_hbm.at[tid].at[gsl], dst_v)

                def body(j, _):
                    pltpu.async_copy(table.at[src_v.at[j]], rows_v, sem).wait()
                    pltpu.sync_copy(rows_v, acc.at[dst_v.at[j]], add=True)
                    return 0

                lax.fori_loop(0, G, body, 0)
                return 0

            lax.fori_loop(0, rows // G, group, 0)
            plsc.subcore_barrier()
            pltpu.sync_copy(acc.at[nsl], out.at[nsl])

        @pl.when(c == 0)
        def _():
            run(table_a, out_a)

        @pl.when(c == 1)
        def _():
            run(table_b, out_b)

    return agg


_agg_channel_split = _make_agg(False)
_agg_edge_split = _make_agg(True)


def _tc1_body(x_ref, w_ref, d0_ref, d1_ref, ha_ref, hb_ref):
    dinv = lax.rsqrt(d0_ref[...] + d1_ref[...] + 1.0)
    h = jnp.dot(x_ref[...], w_ref[...], preferred_element_type=_f32) * dinv
    ha_ref[...] = h[:, :D]
    hb_ref[...] = h[:, D:]


def _tc2_body(a_ref, b_ref, d0_ref, d1_ref, b1a_ref, b1b_ref,
              w2a_ref, w2b_ref, out_ref):
    dinv = lax.rsqrt(d0_ref[...] + d1_ref[...] + 1.0)
    z0 = jnp.maximum(a_ref[...] * dinv + b1a_ref[...], 0.0)
    z1 = jnp.maximum(b_ref[...] * dinv + b1b_ref[...], 0.0)
    h = (jnp.dot(z0, w2a_ref[...], preferred_element_type=_f32)
         + jnp.dot(z1, w2b_ref[...], preferred_element_type=_f32))
    out_ref[...] = h * dinv


def _tc3_body(aa_ref, ab_ref, h2_ref, d0_ref, d1_ref, b2_ref, out_ref):
    dinv = lax.rsqrt(d0_ref[...] + d1_ref[...] + 1.0)
    out_ref[...] = (aa_ref[...] + ab_ref[...] - h2_ref[...]) * dinv + b2_ref[...]


_BN = 640


def _row_spec(width):
    return pl.BlockSpec((_BN, width), lambda i: (i, 0))


def _full_spec(shape):
    return pl.BlockSpec(shape, lambda i: (0,) * len(shape))


def _tc1(x_pad, w1, d0, d1):
    return pl.pallas_call(
        _tc1_body,
        grid=(NPAD // _BN,),
        in_specs=[_row_spec(D), _full_spec((D, 2 * D)),
                  _row_spec(1), _row_spec(1)],
        out_specs=[_row_spec(D), _row_spec(D)],
        out_shape=[jax.ShapeDtypeStruct((NPAD, D), _f32)] * 2,
    )(x_pad, w1, d0, d1)


def _tc2(a, b, d0, d1, b1a, b1b, w2a, w2b):
    return pl.pallas_call(
        _tc2_body,
        grid=(NPAD // _BN,),
        in_specs=[_row_spec(D), _row_spec(D), _row_spec(1), _row_spec(1),
                  _full_spec((1, D)), _full_spec((1, D)),
                  _full_spec((D, D)), _full_spec((D, D))],
        out_specs=_row_spec(D),
        out_shape=jax.ShapeDtypeStruct((NPAD, D), _f32),
    )(a, b, d0, d1, b1a, b1b, w2a, w2b)


def _tc3(aa, ab, h2, d0, d1, b2):
    return pl.pallas_call(
        _tc3_body,
        grid=(NPAD // _BN,),
        in_specs=[_row_spec(D), _row_spec(D), _row_spec(D),
                  _row_spec(1), _row_spec(1), _full_spec((1, D))],
        out_specs=_row_spec(D),
        out_shape=jax.ShapeDtypeStruct((NPAD, D), _f32),
    )(aa, ab, h2, d0, d1, b2)


def kernel(x, edge_index, W1, b1, W2, b2):
    src32 = edge_index[0].reshape(NC * NS, E // (NC * NS * CH), CH)
    dst32 = edge_index[1].reshape(NC * NS, E // (NC * NS * CH), CH)
    src16 = edge_index[0].reshape(NS, E // (NS * CH), CH)
    dst16 = edge_index[1].reshape(NS, E // (NS * CH), CH)
    x_pad = jnp.pad(x, ((0, NPAD - N), (0, 0)))

    deg_a, deg_b = _deg_kernel(dst32)
    d0 = deg_a[:, None]
    d1 = deg_b[:, None]

    h1a, h1b = _tc1(x_pad, W1, d0, d1)
    agg1a, agg1b = _agg_channel_split(src16, dst16, h1a, h1b)

    h2 = _tc2(agg1a, agg1b, d0, d1,
              b1[:D].reshape(1, D), b1[D:].reshape(1, D),
              W2[:D], W2[D:])
    agg2a, agg2b = _agg_edge_split(src32, dst32, h2, h2)

    out = _tc3(agg2a, agg2b, h2, d0, d1, b2.reshape(1, D))
    return out[:N]


# speedup vs baseline: 25.9531x; 1.4975x over previous
import functools

import jax
import jax.numpy as jnp
from jax import lax
from jax.experimental import pallas as pl
from jax.experimental.pallas import tpu as pltpu
from jax.experimental.pallas import tpu_sc as plsc

N = 10000
NPAD = 10240
E = 320000
D = 128
CH = 125
G = 8
NC = 2
NS = 16
RPT = NPAD // NS

_mesh = plsc.VectorSubcoreMesh(core_axis_name="c", subcore_axis_name="s")
_f32 = jnp.float32


@functools.partial(
    pl.kernel,
    mesh=_mesh,
    out_type=[jax.ShapeDtypeStruct((NPAD,), _f32),
              jax.ShapeDtypeStruct((NPAD,), _f32)],
    scratch_types=[
        pltpu.VMEM((E // (NC * NS * CH), CH), jnp.int32),
        pltpu.VMEM((128,), _f32),
        pltpu.VMEM((RPT,), _f32),
        pltpu.VMEM_SHARED((NPAD,), _f32),
    ],
)
def _deg_kernel(dst_hbm, deg_a, deg_b, dst_v, ones_v, zero_v, acc):
    c = lax.axis_index("c")
    s = lax.axis_index("s")
    rows = E // (NC * NS * CH)
    wid = s * NC + c

    def fill_ones(i, _):
        ones_v[pl.ds(i * 16, 16)] = jnp.ones((16,), _f32)
        return 0

    lax.fori_loop(0, 8, fill_ones, 0)

    def fill_zero(i, _):
        zero_v[pl.ds(i * 16, 16)] = jnp.zeros((16,), _f32)
        return 0

    lax.fori_loop(0, RPT // 16, fill_zero, 0)

    nsl = pl.ds(s * RPT, RPT)
    pltpu.sync_copy(zero_v, acc.at[nsl])
    pltpu.sync_copy(dst_hbm.at[wid], dst_v)
    plsc.subcore_barrier()

    def body(j, _):
        pltpu.sync_copy(ones_v.at[pl.ds(0, CH)], acc.at[dst_v.at[j]], add=True)
        return 0

    lax.fori_loop(0, rows, body, 0)
    plsc.subcore_barrier()

    @pl.when(c == 0)
    def _():
        pltpu.sync_copy(acc.at[nsl], deg_a.at[nsl])

    @pl.when(c == 1)
    def _():
        pltpu.sync_copy(acc.at[nsl], deg_b.at[nsl])


def _make_agg(split_edges_by_core: bool):
    rows = E // (NC * NS * CH) if split_edges_by_core else E // (NS * CH)
    ngrp = rows // G

    @functools.partial(
        pl.kernel,
        mesh=_mesh,
        out_type=[jax.ShapeDtypeStruct((NPAD, D), _f32),
                  jax.ShapeDtypeStruct((NPAD, D), _f32)],
        scratch_types=[
            pltpu.VMEM((2, G, CH), jnp.int32),
            pltpu.VMEM((2, G, CH), jnp.int32),
            pltpu.VMEM((2, CH, D), _f32),
            pltpu.VMEM_SHARED((NPAD, D), _f32),
            pltpu.SemaphoreType.DMA,
            pltpu.SemaphoreType.DMA,
        ],
    )
    def agg(src_hbm, dst_hbm, table_a, table_b, out_a, out_b,
            src_v, dst_v, rows_v, acc, gsem, isem):
        c = lax.axis_index("c")
        s = lax.axis_index("s")
        if split_edges_by_core:
            tid = s * NC + c
        else:
            tid = s
        nsl = pl.ds(s * RPT, RPT)

        def run(table, out):
            src_t = src_hbm.at[tid]
            dst_t = dst_hbm.at[tid]

            def gather(b, r, buf):
                return pltpu.make_async_copy(
                    table.at[src_v.at[b].at[r]], rows_v.at[buf], gsem)

            def stage(g, b):
                gsl = pl.ds(pl.multiple_of(g * G, G), G)
                return (pltpu.make_async_copy(src_t.at[gsl], src_v.at[b], isem),
                        pltpu.make_async_copy(dst_t.at[gsl], dst_v.at[b], isem))

            pltpu.sync_copy(table.at[nsl], acc.at[nsl])
            plsc.subcore_barrier()
            for d in stage(0, 0):
                d.start()
                d.wait()

            def group(g, _):
                b = lax.rem(g, 2)
                nxt = 1 - b

                @pl.when(g + 1 < ngrp)
                def _():
                    for d in stage(g + 1, nxt):
                        d.start()

                gather(b, 0, 0).start()
                for r in range(1, G):
                    gather(b, r, r % 2).start()
                    gather(b, r - 1, (r - 1) % 2).wait()
                    pltpu.sync_copy(rows_v.at[(r - 1) % 2],
                                    acc.at[dst_v.at[b].at[r - 1]], add=True)
                gather(b, G - 1, (G - 1) % 2).wait()
                pltpu.sync_copy(rows_v.at[(G - 1) % 2],
                                acc.at[dst_v.at[b].at[G - 1]], add=True)

                @pl.when(g + 1 < ngrp)
                def _():
                    for d in stage(g + 1, nxt):
                        d.wait()

                return 0

            lax.fori_loop(0, ngrp, group, 0)
            plsc.subcore_barrier()
            pltpu.sync_copy(acc.at[nsl], out.at[nsl])

        @pl.when(c == 0)
        def _():
            run(table_a, out_a)

        @pl.when(c == 1)
        def _():
            run(table_b, out_b)

    return agg


_agg_channel_split = _make_agg(False)
_agg_edge_split = _make_agg(True)


def _tc1_body(x_ref, w_ref, d0_ref, d1_ref, ha_ref, hb_ref):
    dinv = lax.rsqrt(d0_ref[...] + d1_ref[...] + 1.0)
    h = jnp.dot(x_ref[...], w_ref[...], preferred_element_type=_f32) * dinv
    ha_ref[...] = h[:, :D]
    hb_ref[...] = h[:, D:]


def _tc2_body(a_ref, b_ref, d0_ref, d1_ref, b1a_ref, b1b_ref,
              w2a_ref, w2b_ref, out_ref):
    dinv = lax.rsqrt(d0_ref[...] + d1_ref[...] + 1.0)
    z0 = jnp.maximum(a_ref[...] * dinv + b1a_ref[...], 0.0)
    z1 = jnp.maximum(b_ref[...] * dinv + b1b_ref[...], 0.0)
    h = (jnp.dot(z0, w2a_ref[...], preferred_element_type=_f32)
         + jnp.dot(z1, w2b_ref[...], preferred_element_type=_f32))
    out_ref[...] = h * dinv


def _tc3_body(aa_ref, ab_ref, h2_ref, d0_ref, d1_ref, b2_ref, out_ref):
    dinv = lax.rsqrt(d0_ref[...] + d1_ref[...] + 1.0)
    out_ref[...] = (aa_ref[...] + ab_ref[...] - h2_ref[...]) * dinv + b2_ref[...]


_BN = 640


def _row_spec(width):
    return pl.BlockSpec((_BN, width), lambda i: (i, 0))


def _full_spec(shape):
    return pl.BlockSpec(shape, lambda i: (0,) * len(shape))


def _tc1(x_pad, w1, d0, d1):
    return pl.pallas_call(
        _tc1_body,
        grid=(NPAD // _BN,),
        in_specs=[_row_spec(D), _full_spec((D, 2 * D)),
                  _row_spec(1), _row_spec(1)],
        out_specs=[_row_spec(D), _row_spec(D)],
        out_shape=[jax.ShapeDtypeStruct((NPAD, D), _f32)] * 2,
    )(x_pad, w1, d0, d1)


def _tc2(a, b, d0, d1, b1a, b1b, w2a, w2b):
    return pl.pallas_call(
        _tc2_body,
        grid=(NPAD // _BN,),
        in_specs=[_row_spec(D), _row_spec(D), _row_spec(1), _row_spec(1),
                  _full_spec((1, D)), _full_spec((1, D)),
                  _full_spec((D, D)), _full_spec((D, D))],
        out_specs=_row_spec(D),
        out_shape=jax.ShapeDtypeStruct((NPAD, D), _f32),
    )(a, b, d0, d1, b1a, b1b, w2a, w2b)


def _tc3(aa, ab, h2, d0, d1, b2):
    return pl.pallas_call(
        _tc3_body,
        grid=(NPAD // _BN,),
        in_specs=[_row_spec(D), _row_spec(D), _row_spec(D),
                  _row_spec(1), _row_spec(1), _full_spec((1, D))],
        out_specs=_row_spec(D),
        out_shape=jax.ShapeDtypeStruct((NPAD, D), _f32),
    )(aa, ab, h2, d0, d1, b2)


def kernel(x, edge_index, W1, b1, W2, b2):
    src32 = edge_index[0].reshape(NC * NS, E // (NC * NS * CH), CH)
    dst32 = edge_index[1].reshape(NC * NS, E // (NC * NS * CH), CH)
    src16 = edge_index[0].reshape(NS, E // (NS * CH), CH)
    dst16 = edge_index[1].reshape(NS, E // (NS * CH), CH)
    x_pad = jnp.pad(x, ((0, NPAD - N), (0, 0)))

    deg_a, deg_b = _deg_kernel(dst32)
    d0 = deg_a[:, None]
    d1 = deg_b[:, None]

    h1a, h1b = _tc1(x_pad, W1, d0, d1)
    agg1a, agg1b = _agg_channel_split(src16, dst16, h1a, h1b)

    h2 = _tc2(agg1a, agg1b, d0, d1,
              b1[:D].reshape(1, D), b1[D:].reshape(1, D),
              W2[:D], W2[D:])
    agg2a, agg2b = _agg_edge_split(src32, dst32, h2, h2)

    out = _tc3(agg2a, agg2b, h2, d0, d1, b2.reshape(1, D))
    return out[:N]


# speedup vs baseline: 29.4275x; 1.1339x over previous
import functools

import jax
import jax.numpy as jnp
from jax import lax
from jax.experimental import pallas as pl
from jax.experimental.pallas import tpu as pltpu
from jax.experimental.pallas import tpu_sc as plsc

N = 10000
NPAD = 10240
E = 320000
D = 128
CH = 100
CHD = 125
G = 10
NBUF = 3
NC = 2
NS = 16
RPT = NPAD // NS

_mesh = plsc.VectorSubcoreMesh(core_axis_name="c", subcore_axis_name="s")
_f32 = jnp.float32


@functools.partial(
    pl.kernel,
    mesh=_mesh,
    out_type=[jax.ShapeDtypeStruct((NPAD,), _f32),
              jax.ShapeDtypeStruct((NPAD,), _f32)],
    scratch_types=[
        pltpu.VMEM((E // (NC * NS * CHD), CHD), jnp.int32),
        pltpu.VMEM((128,), _f32),
        pltpu.VMEM((RPT,), _f32),
        pltpu.VMEM_SHARED((NPAD,), _f32),
    ],
)
def _deg_kernel(dst_hbm, deg_a, deg_b, dst_v, ones_v, zero_v, acc):
    c = lax.axis_index("c")
    s = lax.axis_index("s")
    rows = E // (NC * NS * CHD)
    wid = s * NC + c

    def fill_ones(i, _):
        ones_v[pl.ds(i * 16, 16)] = jnp.ones((16,), _f32)
        return 0

    lax.fori_loop(0, 8, fill_ones, 0)

    def fill_zero(i, _):
        zero_v[pl.ds(i * 16, 16)] = jnp.zeros((16,), _f32)
        return 0

    lax.fori_loop(0, RPT // 16, fill_zero, 0)

    nsl = pl.ds(s * RPT, RPT)
    pltpu.sync_copy(zero_v, acc.at[nsl])
    pltpu.sync_copy(dst_hbm.at[wid], dst_v)
    plsc.subcore_barrier()

    def body(j, _):
        pltpu.sync_copy(ones_v.at[pl.ds(0, CHD)], acc.at[dst_v.at[j]], add=True)
        return 0

    lax.fori_loop(0, rows, body, 0)
    plsc.subcore_barrier()

    @pl.when(c == 0)
    def _():
        pltpu.sync_copy(acc.at[nsl], deg_a.at[nsl])

    @pl.when(c == 1)
    def _():
        pltpu.sync_copy(acc.at[nsl], deg_b.at[nsl])


def _make_agg(split_edges_by_core: bool):
    rows = E // (NC * NS * CH) if split_edges_by_core else E // (NS * CH)

    @functools.partial(
        pl.kernel,
        mesh=_mesh,
        out_type=[jax.ShapeDtypeStruct((NPAD, D), _f32),
                  jax.ShapeDtypeStruct((NPAD, D), _f32)],
        scratch_types=[
            pltpu.VMEM((2 * G, 1, CH), jnp.int32),
            pltpu.VMEM((2 * G, 1, CH), jnp.int32),
            pltpu.VMEM((NBUF, CH, D), _f32),
            pltpu.VMEM_SHARED((NPAD, D), _f32),
            pltpu.SemaphoreType.DMA,
            pltpu.SemaphoreType.DMA,
            pltpu.SemaphoreType.DMA,
        ],
    )
    def agg(src_hbm, dst_hbm, table_a, table_b, out_a, out_b,
            src_v, dst_v, rows_v, acc, gsem, ssem, isem):
        c = lax.axis_index("c")
        s = lax.axis_index("s")
        if split_edges_by_core:
            tid = s * NC + c
        else:
            tid = s
        nsl = pl.ds(s * RPT, RPT)

        def run(table, out):
            src_t = src_hbm.at[tid]
            dst_t = dst_hbm.at[tid]

            def stage(g):
                half = lax.rem(g, 2) * G
                gsl = pl.ds(g * G, G)
                vsl = pl.ds(half, G)
                return (pltpu.make_async_copy(src_t.at[gsl], src_v.at[vsl], isem),
                        pltpu.make_async_copy(dst_t.at[gsl], dst_v.at[vsl], isem))

            def gather(j):
                slot = lax.rem(j, 2 * G)
                return pltpu.make_async_copy(
                    table.at[src_v.at[slot].at[0]],
                    rows_v.at[lax.rem(j, NBUF)], gsem)

            def scatter_start(j):
                slot = lax.rem(j, 2 * G)
                pltpu.async_copy(rows_v.at[lax.rem(j, NBUF)],
                                 acc.at[dst_v.at[slot].at[0]], ssem, add=True)

            def scatter_wait(j):
                slot = lax.rem(j, 2 * G)
                pltpu.make_async_copy(rows_v.at[lax.rem(j, NBUF)],
                                      acc.at[dst_v.at[slot].at[0]], ssem).wait()

            for d in stage(0):
                d.start()
            pltpu.sync_copy(table.at[nsl], acc.at[nsl])
            for d in stage(0):
                d.wait()
            plsc.subcore_barrier()
            gather(0).start()

            def body(j, _):
                @pl.when(jnp.logical_and(lax.rem(j, G) == 0, j + G < rows))
                def _():
                    for d in stage(j // G + 1):
                        d.start()

                @pl.when(j + 1 < rows)
                def _():
                    @pl.when(j >= NBUF - 1)
                    def _():
                        scatter_wait(j - (NBUF - 1))

                    @pl.when(lax.rem(j + 1, G) == 0)
                    def _():
                        for d in stage((j + 1) // G):
                            d.wait()

                    gather(j + 1).start()

                gather(j).wait()
                scatter_start(j)
                return 0

            lax.fori_loop(0, rows, body, 0)
            for k in range(NBUF):
                scatter_wait(rows - NBUF + k)
            plsc.subcore_barrier()
            pltpu.sync_copy(acc.at[nsl], out.at[nsl])

        @pl.when(c == 0)
        def _():
            run(table_a, out_a)

        @pl.when(c == 1)
        def _():
            run(table_b, out_b)

    return agg


_agg_channel_split = _make_agg(False)
_agg_edge_split = _make_agg(True)


def _tc1_body(x_ref, w_ref, d0_ref, d1_ref, ha_ref, hb_ref):
    dinv = lax.rsqrt(d0_ref[...] + d1_ref[...] + 1.0)
    h = jnp.dot(x_ref[...], w_ref[...], preferred_element_type=_f32) * dinv
    ha_ref[...] = h[:, :D]
    hb_ref[...] = h[:, D:]


def _tc2_body(a_ref, b_ref, d0_ref, d1_ref, b1a_ref, b1b_ref,
              w2a_ref, w2b_ref, out_ref):
    dinv = lax.rsqrt(d0_ref[...] + d1_ref[...] + 1.0)
    z0 = jnp.maximum(a_ref[...] * dinv + b1a_ref[...], 0.0)
    z1 = jnp.maximum(b_ref[...] * dinv + b1b_ref[...], 0.0)
    h = (jnp.dot(z0, w2a_ref[...], preferred_element_type=_f32)
         + jnp.dot(z1, w2b_ref[...], preferred_element_type=_f32))
    out_ref[...] = h * dinv


def _tc3_body(aa_ref, ab_ref, h2_ref, d0_ref, d1_ref, b2_ref, out_ref):
    dinv = lax.rsqrt(d0_ref[...] + d1_ref[...] + 1.0)
    out_ref[...] = (aa_ref[...] + ab_ref[...] - h2_ref[...]) * dinv + b2_ref[...]


_BN = 640


def _row_spec(width):
    return pl.BlockSpec((_BN, width), lambda i: (i, 0))


def _full_spec(shape):
    return pl.BlockSpec(shape, lambda i: (0,) * len(shape))


def _tc1(x_pad, w1, d0, d1):
    return pl.pallas_call(
        _tc1_body,
        grid=(NPAD // _BN,),
        in_specs=[_row_spec(D), _full_spec((D, 2 * D)),
                  _row_spec(1), _row_spec(1)],
        out_specs=[_row_spec(D), _row_spec(D)],
        out_shape=[jax.ShapeDtypeStruct((NPAD, D), _f32)] * 2,
    )(x_pad, w1, d0, d1)


def _tc2(a, b, d0, d1, b1a, b1b, w2a, w2b):
    return pl.pallas_call(
        _tc2_body,
        grid=(NPAD // _BN,),
        in_specs=[_row_spec(D), _row_spec(D), _row_spec(1), _row_spec(1),
                  _full_spec((1, D)), _full_spec((1, D)),
                  _full_spec((D, D)), _full_spec((D, D))],
        out_specs=_row_spec(D),
        out_shape=jax.ShapeDtypeStruct((NPAD, D), _f32),
    )(a, b, d0, d1, b1a, b1b, w2a, w2b)


def _tc3(aa, ab, h2, d0, d1, b2):
    return pl.pallas_call(
        _tc3_body,
        grid=(NPAD // _BN,),
        in_specs=[_row_spec(D), _row_spec(D), _row_spec(D),
                  _row_spec(1), _row_spec(1), _full_spec((1, D))],
        out_specs=_row_spec(D),
        out_shape=jax.ShapeDtypeStruct((NPAD, D), _f32),
    )(aa, ab, h2, d0, d1, b2)


def kernel(x, edge_index, W1, b1, W2, b2):
    src32 = edge_index[0].reshape(NC * NS, E // (NC * NS * CH), 1, CH)
    dst32 = edge_index[1].reshape(NC * NS, E // (NC * NS * CH), 1, CH)
    src16 = edge_index[0].reshape(NS, E // (NS * CH), 1, CH)
    dst16 = edge_index[1].reshape(NS, E // (NS * CH), 1, CH)
    dst_deg = edge_index[1].reshape(NC * NS, E // (NC * NS * CHD), CHD)
    x_pad = jnp.pad(x, ((0, NPAD - N), (0, 0)))

    deg_a, deg_b = _deg_kernel(dst_deg)
    d0 = deg_a[:, None]
    d1 = deg_b[:, None]

    h1a, h1b = _tc1(x_pad, W1, d0, d1)
    agg1a, agg1b = _agg_channel_split(src16, dst16, h1a, h1b)

    h2 = _tc2(agg1a, agg1b, d0, d1,
              b1[:D].reshape(1, D), b1[D:].reshape(1, D),
              W2[:D], W2[D:])
    agg2a, agg2b = _agg_edge_split(src32, dst32, h2, h2)

    out = _tc3(agg2a, agg2b, h2, d0, d1, b2.reshape(1, D))
    return out[:N]


# speedup vs baseline: 37.1847x; 1.2636x over previous
import functools

import jax
import jax.numpy as jnp
from jax import lax
from jax.experimental import pallas as pl
from jax.experimental.pallas import tpu as pltpu
from jax.experimental.pallas import tpu_sc as plsc

N = 10000
NPAD = 10240
E = 320000
D = 128
CH = 100
CHD = 125
G = 10
NBUF = 3
NC = 2
NS = 16
RPT = NPAD // NS

_mesh = plsc.VectorSubcoreMesh(core_axis_name="c", subcore_axis_name="s")
_f32 = jnp.float32


@functools.partial(
    pl.kernel,
    mesh=_mesh,
    out_type=[jax.ShapeDtypeStruct((NPAD,), _f32),
              jax.ShapeDtypeStruct((NPAD,), _f32)],
    scratch_types=[
        pltpu.VMEM((E // (NC * NS * CHD), CHD), jnp.int32),
        pltpu.VMEM((128,), _f32),
        pltpu.VMEM((RPT,), _f32),
        pltpu.VMEM_SHARED((NPAD,), _f32),
    ],
)
def _deg_kernel(dst_hbm, deg_a, deg_b, dst_v, ones_v, zero_v, acc):
    c = lax.axis_index("c")
    s = lax.axis_index("s")
    rows = E // (NC * NS * CHD)
    wid = s * NC + c

    def fill_ones(i, _):
        ones_v[pl.ds(i * 16, 16)] = jnp.ones((16,), _f32)
        return 0

    lax.fori_loop(0, 8, fill_ones, 0)

    def fill_zero(i, _):
        zero_v[pl.ds(i * 16, 16)] = jnp.zeros((16,), _f32)
        return 0

    lax.fori_loop(0, RPT // 16, fill_zero, 0)

    nsl = pl.ds(s * RPT, RPT)
    pltpu.sync_copy(zero_v, acc.at[nsl])
    pltpu.sync_copy(dst_hbm.at[wid], dst_v)
    plsc.subcore_barrier()

    def body(j, _):
        pltpu.sync_copy(ones_v.at[pl.ds(0, CHD)], acc.at[dst_v.at[j]], add=True)
        return 0

    lax.fori_loop(0, rows, body, 0)
    plsc.subcore_barrier()

    @pl.when(c == 0)
    def _():
        pltpu.sync_copy(acc.at[nsl], deg_a.at[nsl])

    @pl.when(c == 1)
    def _():
        pltpu.sync_copy(acc.at[nsl], deg_b.at[nsl])


def _make_agg(split_edges_by_core: bool):
    rows = E // (NC * NS * CH) if split_edges_by_core else E // (NS * CH)

    @functools.partial(
        pl.kernel,
        mesh=_mesh,
        out_type=[jax.ShapeDtypeStruct((NPAD, D), _f32),
                  jax.ShapeDtypeStruct((NPAD, D), _f32)],
        scratch_types=[
            pltpu.VMEM((2 * G, 1, CH), jnp.int32),
            pltpu.VMEM((2 * G, 1, CH), jnp.int32),
            pltpu.VMEM((NBUF, CH, D), _f32),
            pltpu.VMEM_SHARED((NPAD, D), _f32),
            pltpu.SemaphoreType.DMA,
            pltpu.SemaphoreType.DMA,
            pltpu.SemaphoreType.DMA,
        ],
    )
    def agg(src_hbm, dst_hbm, table_a, table_b, out_a, out_b,
            src_v, dst_v, rows_v, acc, gsem, ssem, isem):
        c = lax.axis_index("c")
        s = lax.axis_index("s")
        if split_edges_by_core:
            tid = s * NC + c
        else:
            tid = s
        nsl = pl.ds(s * RPT, RPT)

        def run(table, out):
            src_t = src_hbm.at[tid]
            dst_t = dst_hbm.at[tid]

            def stage(g):
                half = lax.rem(g, 2) * G
                gsl = pl.ds(g * G, G)
                vsl = pl.ds(half, G)
                return (pltpu.make_async_copy(src_t.at[gsl], src_v.at[vsl], isem),
                        pltpu.make_async_copy(dst_t.at[gsl], dst_v.at[vsl], isem))

            def gather(j):
                slot = lax.rem(j, 2 * G)
                return pltpu.make_async_copy(
                    table.at[src_v.at[slot].at[0]],
                    rows_v.at[lax.rem(j, NBUF)], gsem)

            def scatter_start(j):
                slot = lax.rem(j, 2 * G)
                pltpu.async_copy(rows_v.at[lax.rem(j, NBUF)],
                                 acc.at[dst_v.at[slot].at[0]], ssem, add=True)

            def scatter_wait(j):
                slot = lax.rem(j, 2 * G)
                pltpu.make_async_copy(rows_v.at[lax.rem(j, NBUF)],
                                      acc.at[dst_v.at[slot].at[0]], ssem).wait()

            for d in stage(0):
                d.start()
            pltpu.sync_copy(table.at[nsl], acc.at[nsl])
            for d in stage(0):
                d.wait()
            plsc.subcore_barrier()
            gather(0).start()

            def body(j, _):
                @pl.when(jnp.logical_and(lax.rem(j, G) == 0, j + G < rows))
                def _():
                    for d in stage(j // G + 1):
                        d.start()

                @pl.when(j + 1 < rows)
                def _():
                    @pl.when(j >= NBUF - 1)
                    def _():
                        scatter_wait(j - (NBUF - 1))

                    @pl.when(lax.rem(j + 1, G) == 0)
                    def _():
                        for d in stage((j + 1) // G):
                            d.wait()

                    gather(j + 1).start()

                gather(j).wait()
                scatter_start(j)
                return 0

            lax.fori_loop(0, rows, body, 0)
            for k in range(NBUF):
                scatter_wait(rows - NBUF + k)
            plsc.subcore_barrier()
            pltpu.sync_copy(acc.at[nsl], out.at[nsl])

        @pl.when(c == 0)
        def _():
            run(table_a, out_a)

        @pl.when(c == 1)
        def _():
            run(table_b, out_b)

    return agg


_agg_edge_split = _make_agg(True)


def _tca_body(x_ref, d0_ref, d1_ref, out_ref):
    dinv = lax.rsqrt(d0_ref[...] + d1_ref[...] + 1.0)
    out_ref[...] = x_ref[...] * dinv


def _tcb_body(aa_ref, ab_ref, xs_ref, d0_ref, d1_ref, b1_ref,
              w1_ref, w2_ref, out_ref):
    dinv = lax.rsqrt(d0_ref[...] + d1_ref[...] + 1.0)
    u = (aa_ref[...] + ab_ref[...] - xs_ref[...]) * dinv
    h1 = jnp.maximum(
        jnp.dot(u, w1_ref[...], preferred_element_type=_f32) + b1_ref[...], 0.0)
    out_ref[...] = jnp.dot(h1, w2_ref[...], preferred_element_type=_f32) * dinv


def _tc3_body(aa_ref, ab_ref, h2_ref, d0_ref, d1_ref, b2_ref, out_ref):
    dinv = lax.rsqrt(d0_ref[...] + d1_ref[...] + 1.0)
    out_ref[...] = (aa_ref[...] + ab_ref[...] - h2_ref[...]) * dinv + b2_ref[...]


_BN = 640


def _row_spec(width):
    return pl.BlockSpec((_BN, width), lambda i: (i, 0))


def _full_spec(shape):
    return pl.BlockSpec(shape, lambda i: (0,) * len(shape))


def _tca(x_pad, d0, d1):
    return pl.pallas_call(
        _tca_body,
        grid=(NPAD // _BN,),
        in_specs=[_row_spec(D), _row_spec(1), _row_spec(1)],
        out_specs=_row_spec(D),
        out_shape=jax.ShapeDtypeStruct((NPAD, D), _f32),
    )(x_pad, d0, d1)


def _tcb(aa, ab, xs, d0, d1, b1, w1, w2):
    return pl.pallas_call(
        _tcb_body,
        grid=(NPAD // _BN,),
        in_specs=[_row_spec(D), _row_spec(D), _row_spec(D),
                  _row_spec(1), _row_spec(1), _full_spec((1, 2 * D)),
                  _full_spec((D, 2 * D)), _full_spec((2 * D, D))],
        out_specs=_row_spec(D),
        out_shape=jax.ShapeDtypeStruct((NPAD, D), _f32),
    )(aa, ab, xs, d0, d1, b1, w1, w2)


def _tc3(aa, ab, h2, d0, d1, b2):
    return pl.pallas_call(
        _tc3_body,
        grid=(NPAD // _BN,),
        in_specs=[_row_spec(D), _row_spec(D), _row_spec(D),
                  _row_spec(1), _row_spec(1), _full_spec((1, D))],
        out_specs=_row_spec(D),
        out_shape=jax.ShapeDtypeStruct((NPAD, D), _f32),
    )(aa, ab, h2, d0, d1, b2)


def kernel(x, edge_index, W1, b1, W2, b2):
    src32 = edge_index[0].reshape(NC * NS, E // (NC * NS * CH), 1, CH)
    dst32 = edge_index[1].reshape(NC * NS, E // (NC * NS * CH), 1, CH)
    dst_deg = edge_index[1].reshape(NC * NS, E // (NC * NS * CHD), CHD)
    x_pad = jnp.pad(x, ((0, NPAD - N), (0, 0)))

    deg_a, deg_b = _deg_kernel(dst_deg)
    d0 = deg_a[:, None]
    d1 = deg_b[:, None]

    xs = _tca(x_pad, d0, d1)
    agg1a, agg1b = _agg_edge_split(src32, dst32, xs, xs)

    h2 = _tcb(agg1a, agg1b, xs, d0, d1, b1.reshape(1, 2 * D), W1, W2)
    agg2a, agg2b = _agg_edge_split(src32, dst32, h2, h2)

    out = _tc3(agg2a, agg2b, h2, d0, d1, b2.reshape(1, D))
    return out[:N]


# speedup vs baseline: 37.2894x; 1.0028x over previous
import functools

import jax
import jax.numpy as jnp
from jax import lax
from jax.experimental import pallas as pl
from jax.experimental.pallas import tpu as pltpu
from jax.experimental.pallas import tpu_sc as plsc

N = 10000
NPAD = 10240
E = 320000
D = 128
CH = 100
CHD = 125
G = 10
NBUF = 3
NC = 2
NS = 16
RPT = NPAD // NS

_mesh = plsc.VectorSubcoreMesh(core_axis_name="c", subcore_axis_name="s")
_f32 = jnp.float32


@functools.partial(
    pl.kernel,
    mesh=_mesh,
    out_type=[jax.ShapeDtypeStruct((NPAD,), _f32),
              jax.ShapeDtypeStruct((NPAD,), _f32)],
    scratch_types=[
        pltpu.VMEM((E // (NC * NS * CHD), CHD), jnp.int32),
        pltpu.VMEM((128,), _f32),
        pltpu.VMEM((RPT,), _f32),
        pltpu.VMEM_SHARED((NPAD,), _f32),
        pltpu.SemaphoreType.DMA,
    ],
)
def _deg_kernel(dst_hbm, deg_a, deg_b, dst_v, ones_v, zero_v, acc, sem):
    c = lax.axis_index("c")
    s = lax.axis_index("s")
    rows = E // (NC * NS * CHD)
    wid = s * NC + c

    def fill_ones(i, _):
        ones_v[pl.ds(i * 16, 16)] = jnp.ones((16,), _f32)
        return 0

    lax.fori_loop(0, 8, fill_ones, 0)

    def fill_zero(i, _):
        zero_v[pl.ds(i * 16, 16)] = jnp.zeros((16,), _f32)
        return 0

    lax.fori_loop(0, RPT // 16, fill_zero, 0)

    nsl = pl.ds(s * RPT, RPT)
    pltpu.sync_copy(zero_v, acc.at[nsl])
    pltpu.sync_copy(dst_hbm.at[wid], dst_v)
    plsc.subcore_barrier()

    def body(j, _):
        pltpu.async_copy(ones_v.at[pl.ds(0, CHD)], acc.at[dst_v.at[j]], sem,
                         add=True)
        return 0

    lax.fori_loop(0, rows, body, 0)

    def drain(j, _):
        pltpu.make_async_copy(ones_v.at[pl.ds(0, CHD)], acc.at[dst_v.at[j]],
                              sem).wait()
        return 0

    lax.fori_loop(0, rows, drain, 0)
    plsc.subcore_barrier()

    @pl.when(c == 0)
    def _():
        pltpu.sync_copy(acc.at[nsl], deg_a.at[nsl])

    @pl.when(c == 1)
    def _():
        pltpu.sync_copy(acc.at[nsl], deg_b.at[nsl])


def _make_agg(split_edges_by_core: bool):
    rows = E // (NC * NS * CH) if split_edges_by_core else E // (NS * CH)

    @functools.partial(
        pl.kernel,
        mesh=_mesh,
        out_type=[jax.ShapeDtypeStruct((NPAD, D), _f32),
                  jax.ShapeDtypeStruct((NPAD, D), _f32)],
        scratch_types=[
            pltpu.VMEM((2 * G, 1, CH), jnp.int32),
            pltpu.VMEM((2 * G, 1, CH), jnp.int32),
            pltpu.VMEM((NBUF, CH, D), _f32),
            pltpu.VMEM_SHARED((NPAD, D), _f32),
            pltpu.SemaphoreType.DMA,
            pltpu.SemaphoreType.DMA,
            pltpu.SemaphoreType.DMA,
        ],
    )
    def agg(src_hbm, dst_hbm, table_a, table_b, out_a, out_b,
            src_v, dst_v, rows_v, acc, gsem, ssem, isem):
        c = lax.axis_index("c")
        s = lax.axis_index("s")
        if split_edges_by_core:
            tid = s * NC + c
        else:
            tid = s
        nsl = pl.ds(s * RPT, RPT)

        def run(table, out):
            src_t = src_hbm.at[tid]
            dst_t = dst_hbm.at[tid]

            def stage(g):
                half = lax.rem(g, 2) * G
                gsl = pl.ds(g * G, G)
                vsl = pl.ds(half, G)
                return (pltpu.make_async_copy(src_t.at[gsl], src_v.at[vsl], isem),
                        pltpu.make_async_copy(dst_t.at[gsl], dst_v.at[vsl], isem))

            def gather(j):
                slot = lax.rem(j, 2 * G)
                return pltpu.make_async_copy(
                    table.at[src_v.at[slot].at[0]],
                    rows_v.at[lax.rem(j, NBUF)], gsem)

            def scatter_start(j):
                slot = lax.rem(j, 2 * G)
                pltpu.async_copy(rows_v.at[lax.rem(j, NBUF)],
                                 acc.at[dst_v.at[slot].at[0]], ssem, add=True)

            def scatter_wait(j):
                slot = lax.rem(j, 2 * G)
                pltpu.make_async_copy(rows_v.at[lax.rem(j, NBUF)],
                                      acc.at[dst_v.at[slot].at[0]], ssem).wait()

            for d in stage(0):
                d.start()
            pltpu.sync_copy(table.at[nsl], acc.at[nsl])
            for d in stage(0):
                d.wait()
            plsc.subcore_barrier()
            gather(0).start()

            def body(j, _):
                @pl.when(jnp.logical_and(lax.rem(j, G) == 0, j + G < rows))
                def _():
                    for d in stage(j // G + 1):
                        d.start()

                @pl.when(j + 1 < rows)
                def _():
                    @pl.when(j >= NBUF - 1)
                    def _():
                        scatter_wait(j - (NBUF - 1))

                    @pl.when(lax.rem(j + 1, G) == 0)
                    def _():
                        for d in stage((j + 1) // G):
                            d.wait()

                    gather(j + 1).start()

                gather(j).wait()
                scatter_start(j)
                return 0

            lax.fori_loop(0, rows, body, 0)
            for k in range(NBUF):
                scatter_wait(rows - NBUF + k)
            plsc.subcore_barrier()
            pltpu.sync_copy(acc.at[nsl], out.at[nsl])

        @pl.when(c == 0)
        def _():
            run(table_a, out_a)

        @pl.when(c == 1)
        def _():
            run(table_b, out_b)

    return agg


_agg_edge_split = _make_agg(True)


def _tca_body(x_ref, d0_ref, d1_ref, out_ref):
    dinv = lax.rsqrt(d0_ref[...] + d1_ref[...] + 1.0)
    out_ref[...] = x_ref[...] * dinv


def _tcb_body(aa_ref, ab_ref, xs_ref, d0_ref, d1_ref, b1_ref,
              w1_ref, w2_ref, out_ref):
    dinv = lax.rsqrt(d0_ref[...] + d1_ref[...] + 1.0)
    u = (aa_ref[...] + ab_ref[...] - xs_ref[...]) * dinv
    h1 = jnp.maximum(
        jnp.dot(u, w1_ref[...], preferred_element_type=_f32) + b1_ref[...], 0.0)
    out_ref[...] = jnp.dot(h1, w2_ref[...], preferred_element_type=_f32) * dinv


def _tc3_body(aa_ref, ab_ref, h2_ref, d0_ref, d1_ref, b2_ref, out_ref):
    dinv = lax.rsqrt(d0_ref[...] + d1_ref[...] + 1.0)
    out_ref[...] = ((aa_ref[...] + ab_ref[...] - h2_ref[...]) * dinv
                    + b2_ref[...])


_BN = 640


def _row_spec(width):
    return pl.BlockSpec((_BN, width), lambda i: (i, 0))


def _full_spec(shape):
    return pl.BlockSpec(shape, lambda i: (0,) * len(shape))


def _tca(x_pad, d0, d1):
    return pl.pallas_call(
        _tca_body,
        grid=(NPAD // _BN,),
        in_specs=[_row_spec(D), _row_spec(1), _row_spec(1)],
        out_specs=_row_spec(D),
        out_shape=jax.ShapeDtypeStruct((NPAD, D), _f32),
    )(x_pad, d0, d1)


def _tcb(aa, ab, xs, d0, d1, b1, w1, w2):
    return pl.pallas_call(
        _tcb_body,
        grid=(NPAD // _BN,),
        in_specs=[_row_spec(D), _row_spec(D), _row_spec(D),
                  _row_spec(1), _row_spec(1), _full_spec((1, 2 * D)),
                  _full_spec((D, 2 * D)), _full_spec((2 * D, D))],
        out_specs=_row_spec(D),
        out_shape=jax.ShapeDtypeStruct((NPAD, D), _f32),
    )(aa, ab, xs, d0, d1, b1, w1, w2)


def _tc3(aa, ab, h2, d0, d1, b2):
    return pl.pallas_call(
        _tc3_body,
        grid=(NPAD // _BN,),
        in_specs=[_row_spec(D), _row_spec(D), _row_spec(D),
                  _row_spec(1), _row_spec(1), _full_spec((1, D))],
        out_specs=_row_spec(D),
        out_shape=jax.ShapeDtypeStruct((NPAD, D), _f32),
    )(aa, ab, h2, d0, d1, b2)


def kernel(x, edge_index, W1, b1, W2, b2):
    src32 = edge_index[0].reshape(NC * NS, E // (NC * NS * CH), 1, CH)
    dst32 = edge_index[1].reshape(NC * NS, E // (NC * NS * CH), 1, CH)
    dst_deg = edge_index[1].reshape(NC * NS, E // (NC * NS * CHD), CHD)
    x_pad = jnp.pad(x, ((0, NPAD - N), (0, 0)))

    deg_a, deg_b = _deg_kernel(dst_deg)
    d0 = deg_a[:, None]
    d1 = deg_b[:, None]

    xs = _tca(x_pad, d0, d1)
    agg1a, agg1b = _agg_edge_split(src32, dst32, xs, xs)

    h2 = _tcb(agg1a, agg1b, xs, d0, d1, b1.reshape(1, 2 * D), W1, W2)
    agg2a, agg2b = _agg_edge_split(src32, dst32, h2, h2)

    out = _tc3(agg2a, agg2b, h2, d0, d1, b2.reshape(1, D))
    return out[:N]
